# baseline (speedup 1.0000x reference)
import functools

import jax
import jax.numpy as jnp
from jax import lax
from jax.experimental import pallas as pl
from jax.experimental.pallas import tpu as pltpu
from jax.experimental.pallas import tpu_sc as plsc

V, D, O = 1_000_000, 64, 256
B, L = 4096, 200
NC, NS = 2, 16
NW = NC * NS
NB = B // NW


NCH = V // 128
TV0 = V - 128
CHW = 246
NPAIR = CHW // 2


def _tr_body(tT_hbm, tail_hbm, lin_hbm, in0, in1, out0, out1,
             isem0, isem1, osem0, osem1):
    wid = lax.axis_index("s") * NC + lax.axis_index("c")

    def fire_in(j, buf, sem):
        pltpu.async_copy(tT_hbm.at[:, pl.ds(j * 128, 128)], buf, sem)

    def wait_in(buf, sem):
        pltpu.make_async_copy(tT_hbm.at[:, pl.ds(0, 128)], buf, sem).wait()

    def fire_out(j, buf, sem):
        pltpu.async_copy(buf, lin_hbm.at[pl.ds(j * 128 * D, 128 * D)], sem)

    def wait_out(buf, sem):
        pltpu.make_async_copy(buf, lin_hbm.at[pl.ds(0, 128 * D)], sem).wait()

    idxd = [lax.iota(jnp.int32, 16) + (16 * k) for k in range(4)]
    zero16 = jnp.zeros((16,), jnp.int32)

    def transpose(inb, outb):
        for u in range(128):
            uv = zero16 + u
            for k in range(4):
                vals = plsc.load_gather(inb, [idxd[k], uv])
                outb[pl.ds(u * D + k * 16, 16)] = vals

    def chunk(t):
        return (wid + t * NW) % NCH

    fire_in(chunk(0), in0, isem0)
    fire_in(chunk(1), in1, isem1)

    def lt(i, carry):
        def half(t, inb, outb, isem, osem):
            wait_in(inb, isem)

            @pl.when(i > 0)
            def _():
                wait_out(outb, osem)

            transpose(inb, outb)
            fire_out(chunk(t), outb, osem)
            fire_in(chunk(t + 2), inb, isem)

        half(2 * i, in0, out0, isem0, osem0)
        half(2 * i + 1, in1, out1, isem1, osem1)
        return carry

    lax.fori_loop(0, NPAIR, lt, 0)
    wait_in(in0, isem0)
    wait_in(in1, isem1)
    wait_out(out0, osem0)
    wait_out(out1, osem1)

    @pl.when(wid == 0)
    def _():
        pltpu.sync_copy(tail_hbm, in0)
        transpose(in0, out0)
        pltpu.sync_copy(out0, lin_hbm.at[pl.ds(TV0 * D, 128 * D)])


_transpose = functools.partial(
    pl.kernel,
    mesh=plsc.VectorSubcoreMesh(core_axis_name="c", subcore_axis_name="s"),
    compiler_params=pltpu.CompilerParams(needs_layout_passes=False),
    out_type=jax.ShapeDtypeStruct((V * D,), jnp.float32),
    scratch_types=[
        pltpu.VMEM((D, 128), jnp.float32),
        pltpu.VMEM((D, 128), jnp.float32),
        pltpu.VMEM((128 * D,), jnp.float32),
        pltpu.VMEM((128 * D,), jnp.float32),
        pltpu.SemaphoreType.DMA,
        pltpu.SemaphoreType.DMA,
        pltpu.SemaphoreType.DMA,
        pltpu.SemaphoreType.DMA,
    ],
)(_tr_body)


def _pool_body(tokA_hbm, tokB_hbm, table_hbm, pooled_hbm, tokA_v, tokB_v,
               buf0, buf1, out_v, sem0, sem1):
    wid = lax.axis_index("s") * NC + lax.axis_index("c")
    base = wid * NB
    pltpu.sync_copy(tokA_hbm.at[pl.ds(base, NB)], tokA_v)
    pltpu.sync_copy(tokB_hbm.at[pl.ds(base, NB)], tokB_v)

    def fire(b, buf, sem):
        pltpu.async_copy(table_hbm.at[tokA_v.at[b, pl.ds(0, 128)]],
                         buf.at[pl.ds(0, 128)], sem)
        pltpu.async_copy(table_hbm.at[tokB_v.at[b, pl.ds(0, L - 128)]],
                         buf.at[pl.ds(128, L - 128)], sem)

    def wait(buf, sem):
        pltpu.make_async_copy(table_hbm.at[pl.ds(0, L)], buf, sem).wait()

    zeros = jnp.zeros((16,), jnp.float32)

    def process(b, buf):
        def acc_body(i8, accs):
            t0 = i8 * 8
            for dt in range(8):
                accs = tuple(a + buf[t0 + dt, pl.ds(k * 16, 16)]
                             for k, a in enumerate(accs))
            return accs

        accs = lax.fori_loop(0, L // 8, acc_body, (zeros, zeros, zeros, zeros))
        for k in range(4):
            out_v[pl.ds(b * D + k * 16, 16)] = accs[k]

    fire(0, buf0, sem0)
    fire(1, buf1, sem1)

    def loop_body(i, carry):
        b0 = 2 * i
        wait(buf0, sem0)
        process(b0, buf0)

        @pl.when(i < NB // 2 - 1)
        def _():
            fire(b0 + 2, buf0, sem0)

        wait(buf1, sem1)
        process(b0 + 1, buf1)

        @pl.when(i < NB // 2 - 1)
        def _():
            fire(b0 + 3, buf1, sem1)

        return carry

    lax.fori_loop(0, NB // 2, loop_body, 0)
    pltpu.sync_copy(out_v, pooled_hbm.at[pl.ds(base * D, NB * D)])


_pool = functools.partial(
    pl.kernel,
    mesh=plsc.VectorSubcoreMesh(core_axis_name="c", subcore_axis_name="s"),
    compiler_params=pltpu.CompilerParams(use_tc_tiling_on_sc=False),
    out_type=jax.ShapeDtypeStruct((B * D,), jnp.float32),
    scratch_types=[
        pltpu.VMEM((NB, 128), jnp.int32),
        pltpu.VMEM((NB, L - 128), jnp.int32),
        pltpu.VMEM((L, D), jnp.float32),
        pltpu.VMEM((L, D), jnp.float32),
        pltpu.VMEM((NB * D,), jnp.float32),
        pltpu.SemaphoreType.DMA,
        pltpu.SemaphoreType.DMA,
    ],
)(_pool_body)


def _mlp_body(x_ref, tok_ref, w1_ref, b1_ref, w2_ref, b2_ref, o_ref):
    cnt = jnp.sum((tok_ref[...] != 0).astype(jnp.float32), axis=1,
                  keepdims=True)
    x = x_ref[...] / jnp.maximum(cnt, 1.0)
    h = jnp.dot(x, w1_ref[...], preferred_element_type=jnp.float32)
    h = jnp.maximum(h + b1_ref[...], 0.0)
    p = jnp.dot(h, w2_ref[...], preferred_element_type=jnp.float32)
    p = p + b2_ref[...]
    norm = jnp.sqrt(jnp.sum(p * p, axis=-1, keepdims=True))
    o_ref[...] = p / jnp.maximum(norm, 1e-8)


BLK = 512


def _mlp(summed, tokens, W1, b1, W2, b2):
    return pl.pallas_call(
        _mlp_body,
        out_shape=jax.ShapeDtypeStruct((B, O), jnp.float32),
        grid=(B // BLK,),
        in_specs=[
            pl.BlockSpec((BLK, D), lambda i: (i, 0)),
            pl.BlockSpec((BLK, L), lambda i: (i, 0)),
            pl.BlockSpec((D, O), lambda i: (0, 0)),
            pl.BlockSpec((1, O), lambda i: (0, 0)),
            pl.BlockSpec((O, O), lambda i: (0, 0)),
            pl.BlockSpec((1, O), lambda i: (0, 0)),
        ],
        out_specs=pl.BlockSpec((BLK, O), lambda i: (i, 0)),
    )(summed, tokens, W1, b1, W2, b2)


def kernel(tokens, table, W1, b1, W2, b2):
    tT = table.T
    tail = tT[:, TV0:]
    lin = _transpose(tT, tail).reshape(V, D)
    summed = _pool(tokens[:, :128], tokens[:, 128:], lin).reshape(B, D)
    return _mlp(summed, tokens, W1, b1.reshape(1, O), W2, b2.reshape(1, O))


# speedup vs baseline: 2.4840x; 2.4840x over previous
import functools

import jax
import jax.numpy as jnp
from jax import lax
from jax.experimental import pallas as pl
from jax.experimental.pallas import tpu as pltpu
from jax.experimental.pallas import tpu_sc as plsc

V, D, O = 1_000_000, 64, 256
B, L = 4096, 200
NC, NS = 2, 16
NW = NC * NS
NB = B // NW


NCH = V // 128
TV0 = V - 128
CHW = 246
NPAIR = CHW // 2


def _tr_body(tT_hbm, tail_hbm, lin_hbm, in0, in1, out0, out1,
             isem0, isem1, osem0, osem1):
    wid = lax.axis_index("s") * NC + lax.axis_index("c")

    def fire_in(j, buf, sem):
        pltpu.async_copy(tT_hbm.at[:, pl.ds(j * 128, 128)], buf, sem)

    def wait_in(buf, sem):
        pltpu.make_async_copy(tT_hbm.at[:, pl.ds(0, 128)], buf, sem).wait()

    def fire_out(j, buf, sem):
        pltpu.async_copy(buf, lin_hbm.at[pl.ds(j * 128 * D, 128 * D)], sem)

    def wait_out(buf, sem):
        pltpu.make_async_copy(buf, lin_hbm.at[pl.ds(0, 128 * D)], sem).wait()

    iota16 = lax.iota(jnp.int32, 16)
    dvecs = [iota16 + 16 * db for db in range(4)]
    rots = [(iota16 + s) % 16 for s in range(16)]
    outbs = [rots[s] * D + iota16 for s in range(16)]

    def transpose(inb, outb):
        def ub_body(ub, carry):
            u0 = ub * 16
            c0 = ub * (16 * D)
            for db in range(4):
                for s in range(16):
                    u_vec = rots[s] + u0
                    idx_out = outbs[s] + (c0 + 16 * db)
                    vals = plsc.load_gather(inb, [dvecs[db], u_vec])
                    plsc.store_scatter(outb, [idx_out], vals)
            return carry

        lax.fori_loop(0, 8, ub_body, 0)

    def chunk(t):
        return (wid + t * NW) % NCH

    fire_in(chunk(0), in0, isem0)
    fire_in(chunk(1), in1, isem1)

    def lt(i, carry):
        def half(t, inb, outb, isem, osem):
            wait_in(inb, isem)

            @pl.when(i > 0)
            def _():
                wait_out(outb, osem)

            transpose(inb, outb)
            fire_out(chunk(t), outb, osem)
            fire_in(chunk(t + 2), inb, isem)

        half(2 * i, in0, out0, isem0, osem0)
        half(2 * i + 1, in1, out1, isem1, osem1)
        return carry

    lax.fori_loop(0, NPAIR, lt, 0)
    wait_in(in0, isem0)
    wait_in(in1, isem1)
    wait_out(out0, osem0)
    wait_out(out1, osem1)

    @pl.when(wid == 0)
    def _():
        pltpu.sync_copy(tail_hbm, in0)
        transpose(in0, out0)
        pltpu.sync_copy(out0, lin_hbm.at[pl.ds(TV0 * D, 128 * D)])


_transpose = functools.partial(
    pl.kernel,
    mesh=plsc.VectorSubcoreMesh(core_axis_name="c", subcore_axis_name="s"),
    compiler_params=pltpu.CompilerParams(needs_layout_passes=False),
    out_type=jax.ShapeDtypeStruct((V * D,), jnp.float32),
    scratch_types=[
        pltpu.VMEM((D, 128), jnp.float32),
        pltpu.VMEM((D, 128), jnp.float32),
        pltpu.VMEM((128 * D,), jnp.float32),
        pltpu.VMEM((128 * D,), jnp.float32),
        pltpu.SemaphoreType.DMA,
        pltpu.SemaphoreType.DMA,
        pltpu.SemaphoreType.DMA,
        pltpu.SemaphoreType.DMA,
    ],
)(_tr_body)


def _pool_body(tokA_hbm, tokB_hbm, table_hbm, pooled_hbm, tokA_v, tokB_v,
               buf0, buf1, out_v, sem0, sem1):
    wid = lax.axis_index("s") * NC + lax.axis_index("c")
    base = wid * NB
    pltpu.sync_copy(tokA_hbm.at[pl.ds(base, NB)], tokA_v)
    pltpu.sync_copy(tokB_hbm.at[pl.ds(base, NB)], tokB_v)

    def fire(b, buf, sem):
        pltpu.async_copy(table_hbm.at[tokA_v.at[b, pl.ds(0, 128)]],
                         buf.at[pl.ds(0, 128)], sem)
        pltpu.async_copy(table_hbm.at[tokB_v.at[b, pl.ds(0, L - 128)]],
                         buf.at[pl.ds(128, L - 128)], sem)

    def wait(buf, sem):
        pltpu.make_async_copy(table_hbm.at[pl.ds(0, L)], buf, sem).wait()

    zeros = jnp.zeros((16,), jnp.float32)

    def process(b, buf):
        def acc_body(i8, accs):
            t0 = i8 * 8
            for dt in range(8):
                accs = tuple(a + buf[t0 + dt, pl.ds(k * 16, 16)]
                             for k, a in enumerate(accs))
            return accs

        accs = lax.fori_loop(0, L // 8, acc_body, (zeros, zeros, zeros, zeros))
        for k in range(4):
            out_v[pl.ds(b * D + k * 16, 16)] = accs[k]

    fire(0, buf0, sem0)
    fire(1, buf1, sem1)

    def loop_body(i, carry):
        b0 = 2 * i
        wait(buf0, sem0)
        process(b0, buf0)

        @pl.when(i < NB // 2 - 1)
        def _():
            fire(b0 + 2, buf0, sem0)

        wait(buf1, sem1)
        process(b0 + 1, buf1)

        @pl.when(i < NB // 2 - 1)
        def _():
            fire(b0 + 3, buf1, sem1)

        return carry

    lax.fori_loop(0, NB // 2, loop_body, 0)
    pltpu.sync_copy(out_v, pooled_hbm.at[pl.ds(base * D, NB * D)])


_pool = functools.partial(
    pl.kernel,
    mesh=plsc.VectorSubcoreMesh(core_axis_name="c", subcore_axis_name="s"),
    compiler_params=pltpu.CompilerParams(use_tc_tiling_on_sc=False),
    out_type=jax.ShapeDtypeStruct((B * D,), jnp.float32),
    scratch_types=[
        pltpu.VMEM((NB, 128), jnp.int32),
        pltpu.VMEM((NB, L - 128), jnp.int32),
        pltpu.VMEM((L, D), jnp.float32),
        pltpu.VMEM((L, D), jnp.float32),
        pltpu.VMEM((NB * D,), jnp.float32),
        pltpu.SemaphoreType.DMA,
        pltpu.SemaphoreType.DMA,
    ],
)(_pool_body)


def _mlp_body(x_ref, tok_ref, w1_ref, b1_ref, w2_ref, b2_ref, o_ref):
    cnt = jnp.sum((tok_ref[...] != 0).astype(jnp.float32), axis=1,
                  keepdims=True)
    x = x_ref[...] / jnp.maximum(cnt, 1.0)
    h = jnp.dot(x, w1_ref[...], preferred_element_type=jnp.float32)
    h = jnp.maximum(h + b1_ref[...], 0.0)
    p = jnp.dot(h, w2_ref[...], preferred_element_type=jnp.float32)
    p = p + b2_ref[...]
    norm = jnp.sqrt(jnp.sum(p * p, axis=-1, keepdims=True))
    o_ref[...] = p / jnp.maximum(norm, 1e-8)


BLK = 512


def _mlp(summed, tokens, W1, b1, W2, b2):
    return pl.pallas_call(
        _mlp_body,
        out_shape=jax.ShapeDtypeStruct((B, O), jnp.float32),
        grid=(B // BLK,),
        in_specs=[
            pl.BlockSpec((BLK, D), lambda i: (i, 0)),
            pl.BlockSpec((BLK, L), lambda i: (i, 0)),
            pl.BlockSpec((D, O), lambda i: (0, 0)),
            pl.BlockSpec((1, O), lambda i: (0, 0)),
            pl.BlockSpec((O, O), lambda i: (0, 0)),
            pl.BlockSpec((1, O), lambda i: (0, 0)),
        ],
        out_specs=pl.BlockSpec((BLK, O), lambda i: (i, 0)),
    )(summed, tokens, W1, b1, W2, b2)


def kernel(tokens, table, W1, b1, W2, b2):
    tT = table.T
    tail = tT[:, TV0:]
    lin = _transpose(tT, tail).reshape(V, D)
    summed = _pool(tokens[:, :128], tokens[:, 128:], lin).reshape(B, D)
    return _mlp(summed, tokens, W1, b1.reshape(1, O), W2, b2.reshape(1, O))


# speedup vs baseline: 4.8112x; 1.9369x over previous
import functools

import jax
import jax.numpy as jnp
from jax import lax
from jax.experimental import pallas as pl
from jax.experimental.pallas import tpu as pltpu
from jax.experimental.pallas import tpu_sc as plsc

V, D, O = 1_000_000, 64, 256
B, L = 4096, 200
NC, NS = 2, 16
NW = NC * NS
NB = B // NW


NCH = V // 128
TV0 = V - 128
CHW = 246


def _tr_body(tT_hbm, tail_hbm, lin_hbm, in0, in1, in2, out0, out1, out2,
             isem0, isem1, isem2, osem0, osem1, osem2):
    wid = lax.axis_index("s") * NC + lax.axis_index("c")

    def fire_in(j, buf, sem):
        pltpu.async_copy(tT_hbm.at[:, pl.ds(j * 128, 128)], buf, sem)

    def wait_in(buf, sem):
        pltpu.make_async_copy(tT_hbm.at[:, pl.ds(0, 128)], buf, sem).wait()

    def fire_out(j, buf, sem):
        pltpu.async_copy(buf, lin_hbm.at[pl.ds(j * 128 * D, 128 * D)], sem)

    def wait_out(buf, sem):
        pltpu.make_async_copy(buf, lin_hbm.at[pl.ds(0, 128 * D)], sem).wait()

    iota16 = lax.iota(jnp.int32, 16)
    dvecs = [iota16 + 16 * db for db in range(4)]
    rots = [(iota16 + s) % 16 for s in range(16)]
    outbs = [rots[s] * D + iota16 for s in range(16)]

    def transpose(inb, outb):
        def ub_body(ub, carry):
            u0 = ub * 16
            c0 = ub * (16 * D)
            for db in range(4):
                vals = [plsc.load_gather(inb, [dvecs[db], rots[s] + u0])
                        for s in range(16)]
                for s in range(16):
                    plsc.store_scatter(outb, [outbs[s] + (c0 + 16 * db)],
                                       vals[s])
            return carry

        lax.fori_loop(0, 8, ub_body, 0)

    def chunk(t):
        return (wid + t * NW) % NCH

    fire_in(chunk(0), in0, isem0)
    fire_in(chunk(1), in1, isem1)
    fire_in(chunk(2), in2, isem2)

    def lt(i, carry):
        def third(t, inb, outb, isem, osem):
            wait_in(inb, isem)

            @pl.when(i > 0)
            def _():
                wait_out(outb, osem)

            transpose(inb, outb)
            fire_out(chunk(t), outb, osem)
            fire_in(chunk(t + 3), inb, isem)

        third(3 * i, in0, out0, isem0, osem0)
        third(3 * i + 1, in1, out1, isem1, osem1)
        third(3 * i + 2, in2, out2, isem2, osem2)
        return carry

    lax.fori_loop(0, CHW // 3, lt, 0)
    wait_in(in0, isem0)
    wait_in(in1, isem1)
    wait_in(in2, isem2)
    wait_out(out0, osem0)
    wait_out(out1, osem1)
    wait_out(out2, osem2)

    @pl.when(wid == 0)
    def _():
        pltpu.sync_copy(tail_hbm, in0)
        transpose(in0, out0)
        pltpu.sync_copy(out0, lin_hbm.at[pl.ds(TV0 * D, 128 * D)])


_transpose = functools.partial(
    pl.kernel,
    mesh=plsc.VectorSubcoreMesh(core_axis_name="c", subcore_axis_name="s"),
    compiler_params=pltpu.CompilerParams(needs_layout_passes=False),
    out_type=jax.ShapeDtypeStruct((V * D,), jnp.float32),
    scratch_types=[
        pltpu.VMEM((D, 128), jnp.float32),
        pltpu.VMEM((D, 128), jnp.float32),
        pltpu.VMEM((D, 128), jnp.float32),
        pltpu.VMEM((128 * D,), jnp.float32),
        pltpu.VMEM((128 * D,), jnp.float32),
        pltpu.VMEM((128 * D,), jnp.float32),
        pltpu.SemaphoreType.DMA,
        pltpu.SemaphoreType.DMA,
        pltpu.SemaphoreType.DMA,
        pltpu.SemaphoreType.DMA,
        pltpu.SemaphoreType.DMA,
        pltpu.SemaphoreType.DMA,
    ],
)(_tr_body)


def _pool_body(tokA_hbm, tokB_hbm, table_hbm, pooled_hbm, tokA_v, tokB_v,
               buf0, buf1, out_v, sem0, sem1):
    wid = lax.axis_index("s") * NC + lax.axis_index("c")
    base = wid * NB
    pltpu.sync_copy(tokA_hbm.at[pl.ds(base, NB)], tokA_v)
    pltpu.sync_copy(tokB_hbm.at[pl.ds(base, NB)], tokB_v)

    def fire(b, buf, sem):
        pltpu.async_copy(table_hbm.at[tokA_v.at[b, pl.ds(0, 128)]],
                         buf.at[pl.ds(0, 128)], sem)
        pltpu.async_copy(table_hbm.at[tokB_v.at[b, pl.ds(0, L - 128)]],
                         buf.at[pl.ds(128, L - 128)], sem)

    def wait(buf, sem):
        pltpu.make_async_copy(table_hbm.at[pl.ds(0, L)], buf, sem).wait()

    zeros = jnp.zeros((16,), jnp.float32)

    def process(b, buf):
        def acc_body(i8, accs):
            t0 = i8 * 8
            for dt in range(8):
                accs = tuple(a + buf[t0 + dt, pl.ds(k * 16, 16)]
                             for k, a in enumerate(accs))
            return accs

        accs = lax.fori_loop(0, L // 8, acc_body, (zeros, zeros, zeros, zeros))
        for k in range(4):
            out_v[pl.ds(b * D + k * 16, 16)] = accs[k]

    fire(0, buf0, sem0)
    fire(1, buf1, sem1)

    def loop_body(i, carry):
        b0 = 2 * i
        wait(buf0, sem0)
        process(b0, buf0)

        @pl.when(i < NB // 2 - 1)
        def _():
            fire(b0 + 2, buf0, sem0)

        wait(buf1, sem1)
        process(b0 + 1, buf1)

        @pl.when(i < NB // 2 - 1)
        def _():
            fire(b0 + 3, buf1, sem1)

        return carry

    lax.fori_loop(0, NB // 2, loop_body, 0)
    pltpu.sync_copy(out_v, pooled_hbm.at[pl.ds(base * D, NB * D)])


_pool = functools.partial(
    pl.kernel,
    mesh=plsc.VectorSubcoreMesh(core_axis_name="c", subcore_axis_name="s"),
    compiler_params=pltpu.CompilerParams(use_tc_tiling_on_sc=False),
    out_type=jax.ShapeDtypeStruct((B * D,), jnp.float32),
    scratch_types=[
        pltpu.VMEM((NB, 128), jnp.int32),
        pltpu.VMEM((NB, L - 128), jnp.int32),
        pltpu.VMEM((L, D), jnp.float32),
        pltpu.VMEM((L, D), jnp.float32),
        pltpu.VMEM((NB * D,), jnp.float32),
        pltpu.SemaphoreType.DMA,
        pltpu.SemaphoreType.DMA,
    ],
)(_pool_body)


def _mlp_body(x_ref, tok_ref, w1_ref, b1_ref, w2_ref, b2_ref, o_ref):
    cnt = jnp.sum((tok_ref[...] != 0).astype(jnp.float32), axis=1,
                  keepdims=True)
    x = x_ref[...] / jnp.maximum(cnt, 1.0)
    h = jnp.dot(x, w1_ref[...], preferred_element_type=jnp.float32)
    h = jnp.maximum(h + b1_ref[...], 0.0)
    p = jnp.dot(h, w2_ref[...], preferred_element_type=jnp.float32)
    p = p + b2_ref[...]
    norm = jnp.sqrt(jnp.sum(p * p, axis=-1, keepdims=True))
    o_ref[...] = p / jnp.maximum(norm, 1e-8)


BLK = 512


def _mlp(summed, tokens, W1, b1, W2, b2):
    return pl.pallas_call(
        _mlp_body,
        out_shape=jax.ShapeDtypeStruct((B, O), jnp.float32),
        grid=(B // BLK,),
        in_specs=[
            pl.BlockSpec((BLK, D), lambda i: (i, 0)),
            pl.BlockSpec((BLK, L), lambda i: (i, 0)),
            pl.BlockSpec((D, O), lambda i: (0, 0)),
            pl.BlockSpec((1, O), lambda i: (0, 0)),
            pl.BlockSpec((O, O), lambda i: (0, 0)),
            pl.BlockSpec((1, O), lambda i: (0, 0)),
        ],
        out_specs=pl.BlockSpec((BLK, O), lambda i: (i, 0)),
    )(summed, tokens, W1, b1, W2, b2)


def kernel(tokens, table, W1, b1, W2, b2):
    tT = table.T
    tail = tT[:, TV0:]
    lin = _transpose(tT, tail).reshape(V, D)
    summed = _pool(tokens[:, :128], tokens[:, 128:], lin).reshape(B, D)
    return _mlp(summed, tokens, W1, b1.reshape(1, O), W2, b2.reshape(1, O))


# speedup vs baseline: 5.3857x; 1.1194x over previous
import functools

import jax
import jax.numpy as jnp
from jax import lax
from jax.experimental import pallas as pl
from jax.experimental.pallas import tpu as pltpu
from jax.experimental.pallas import tpu_sc as plsc

V, D, O = 1_000_000, 64, 256
B, L = 4096, 200
NC, NS = 2, 16
NW = NC * NS
NB = B // NW


NCH = V // 128
TV0 = V - 128
CHW = 246


def _tr_body(tT_hbm, tail_hbm, lin_hbm, in0, in1, in2, out0, out1, out2,
             isem0, isem1, isem2, osem0, osem1, osem2):
    wid = lax.axis_index("s") * NC + lax.axis_index("c")

    def fire_in(j, buf, sem):
        pltpu.async_copy(tT_hbm.at[:, pl.ds(j * 128, 128)], buf, sem)

    def wait_in(buf, sem):
        pltpu.make_async_copy(tT_hbm.at[:, pl.ds(0, 128)], buf, sem).wait()

    def fire_out(j, buf, sem):
        pltpu.async_copy(buf, lin_hbm.at[pl.ds(j * 128 * D, 128 * D)], sem)

    def wait_out(buf, sem):
        pltpu.make_async_copy(buf, lin_hbm.at[pl.ds(0, 128 * D)], sem).wait()

    iota16 = lax.iota(jnp.int32, 16)
    dvecs = [iota16 + 16 * db for db in range(4)]
    rots = [(iota16 + s) % 16 for s in range(16)]
    outbs = [rots[s] * D + iota16 for s in range(16)]

    def transpose(inb, outb):
        def ub_body(ub, carry):
            u0 = ub * 16
            c0 = ub * (16 * D)
            for db in range(4):
                vals = [plsc.load_gather(inb, [dvecs[db], rots[s] + u0])
                        for s in range(16)]
                for s in range(16):
                    plsc.store_scatter(outb, [outbs[s] + (c0 + 16 * db)],
                                       vals[s])
            return carry

        lax.fori_loop(0, 8, ub_body, 0)

    def chunk(t):
        return (wid + t * NW) % NCH

    fire_in(chunk(0), in0, isem0)
    fire_in(chunk(1), in1, isem1)
    fire_in(chunk(2), in2, isem2)

    def lt(i, carry):
        def third(t, inb, outb, isem, osem):
            wait_in(inb, isem)

            @pl.when(i > 0)
            def _():
                wait_out(outb, osem)

            transpose(inb, outb)
            fire_out(chunk(t), outb, osem)
            fire_in(chunk(t + 3), inb, isem)

        third(3 * i, in0, out0, isem0, osem0)
        third(3 * i + 1, in1, out1, isem1, osem1)
        third(3 * i + 2, in2, out2, isem2, osem2)
        return carry

    lax.fori_loop(0, CHW // 3, lt, 0)
    wait_in(in0, isem0)
    wait_in(in1, isem1)
    wait_in(in2, isem2)
    wait_out(out0, osem0)
    wait_out(out1, osem1)
    wait_out(out2, osem2)

    @pl.when(wid == 0)
    def _():
        pltpu.sync_copy(tail_hbm, in0)
        transpose(in0, out0)
        pltpu.sync_copy(out0, lin_hbm.at[pl.ds(TV0 * D, 128 * D)])


_transpose = functools.partial(
    pl.kernel,
    mesh=plsc.VectorSubcoreMesh(core_axis_name="c", subcore_axis_name="s"),
    compiler_params=pltpu.CompilerParams(needs_layout_passes=False),
    out_type=jax.ShapeDtypeStruct((V * D,), jnp.float32),
    scratch_types=[
        pltpu.VMEM((D, 128), jnp.float32),
        pltpu.VMEM((D, 128), jnp.float32),
        pltpu.VMEM((D, 128), jnp.float32),
        pltpu.VMEM((128 * D,), jnp.float32),
        pltpu.VMEM((128 * D,), jnp.float32),
        pltpu.VMEM((128 * D,), jnp.float32),
        pltpu.SemaphoreType.DMA,
        pltpu.SemaphoreType.DMA,
        pltpu.SemaphoreType.DMA,
        pltpu.SemaphoreType.DMA,
        pltpu.SemaphoreType.DMA,
        pltpu.SemaphoreType.DMA,
    ],
)(_tr_body)


def _pool_body(tokA_hbm, tokB_hbm, table_hbm, pooled_hbm, tokA_v, tokB_v,
               buf0, buf1, buf2, buf3, out_v, sem0, sem1, sem2, sem3):
    wid = lax.axis_index("s") * NC + lax.axis_index("c")
    base = wid * NB
    pltpu.sync_copy(tokA_hbm.at[pl.ds(base, NB)], tokA_v)
    pltpu.sync_copy(tokB_hbm.at[pl.ds(base, NB)], tokB_v)

    def fire(b, buf, sem):
        pltpu.async_copy(table_hbm.at[tokA_v.at[b, pl.ds(0, 128)]],
                         buf.at[pl.ds(0, 128)], sem)
        pltpu.async_copy(table_hbm.at[tokB_v.at[b, pl.ds(0, L - 128)]],
                         buf.at[pl.ds(128, L - 128)], sem)

    def wait(buf, sem):
        pltpu.make_async_copy(table_hbm.at[pl.ds(0, L)], buf, sem).wait()

    zeros = jnp.zeros((16,), jnp.float32)

    def process(b, buf):
        def acc_body(i8, accs):
            t0 = i8 * 8
            for dt in range(8):
                accs = tuple(a + buf[t0 + dt, pl.ds(k * 16, 16)]
                             for k, a in enumerate(accs))
            return accs

        accs = lax.fori_loop(0, L // 8, acc_body, (zeros, zeros, zeros, zeros))
        for k in range(4):
            out_v[pl.ds(b * D + k * 16, 16)] = accs[k]

    bufs = (buf0, buf1, buf2, buf3)
    sems = (sem0, sem1, sem2, sem3)
    for q in range(4):
        fire(q, bufs[q], sems[q])

    def loop_body(i, carry):
        b0 = 4 * i
        for q in range(4):
            wait(bufs[q], sems[q])
            process(b0 + q, bufs[q])

            @pl.when(i < NB // 4 - 1)
            def _():
                fire(b0 + q + 4, bufs[q], sems[q])

        return carry

    lax.fori_loop(0, NB // 4, loop_body, 0)
    pltpu.sync_copy(out_v, pooled_hbm.at[pl.ds(base * D, NB * D)])


_pool = functools.partial(
    pl.kernel,
    mesh=plsc.VectorSubcoreMesh(core_axis_name="c", subcore_axis_name="s"),
    compiler_params=pltpu.CompilerParams(use_tc_tiling_on_sc=False),
    out_type=jax.ShapeDtypeStruct((B * D,), jnp.float32),
    scratch_types=[
        pltpu.VMEM((NB, 128), jnp.int32),
        pltpu.VMEM((NB, L - 128), jnp.int32),
        pltpu.VMEM((L, D), jnp.float32),
        pltpu.VMEM((L, D), jnp.float32),
        pltpu.VMEM((L, D), jnp.float32),
        pltpu.VMEM((L, D), jnp.float32),
        pltpu.VMEM((NB * D,), jnp.float32),
        pltpu.SemaphoreType.DMA,
        pltpu.SemaphoreType.DMA,
        pltpu.SemaphoreType.DMA,
        pltpu.SemaphoreType.DMA,
    ],
)(_pool_body)


def _mlp_body(x_ref, tok_ref, w1_ref, b1_ref, w2_ref, b2_ref, o_ref):
    cnt = jnp.sum((tok_ref[...] != 0).astype(jnp.float32), axis=1,
                  keepdims=True)
    x = x_ref[...] / jnp.maximum(cnt, 1.0)
    h = jnp.dot(x, w1_ref[...], preferred_element_type=jnp.float32)
    h = jnp.maximum(h + b1_ref[...], 0.0)
    p = jnp.dot(h, w2_ref[...], preferred_element_type=jnp.float32)
    p = p + b2_ref[...]
    norm = jnp.sqrt(jnp.sum(p * p, axis=-1, keepdims=True))
    o_ref[...] = p / jnp.maximum(norm, 1e-8)


BLK = 512


def _mlp(summed, tokens, W1, b1, W2, b2):
    return pl.pallas_call(
        _mlp_body,
        out_shape=jax.ShapeDtypeStruct((B, O), jnp.float32),
        grid=(B // BLK,),
        in_specs=[
            pl.BlockSpec((BLK, D), lambda i: (i, 0)),
            pl.BlockSpec((BLK, L), lambda i: (i, 0)),
            pl.BlockSpec((D, O), lambda i: (0, 0)),
            pl.BlockSpec((1, O), lambda i: (0, 0)),
            pl.BlockSpec((O, O), lambda i: (0, 0)),
            pl.BlockSpec((1, O), lambda i: (0, 0)),
        ],
        out_specs=pl.BlockSpec((BLK, O), lambda i: (i, 0)),
    )(summed, tokens, W1, b1, W2, b2)


def kernel(tokens, table, W1, b1, W2, b2):
    tT = table.T
    tail = tT[:, TV0:]
    lin = _transpose(tT, tail).reshape(V, D)
    summed = _pool(tokens[:, :128], tokens[:, 128:], lin).reshape(B, D)
    return _mlp(summed, tokens, W1, b1.reshape(1, O), W2, b2.reshape(1, O))


# speedup vs baseline: 6.0722x; 1.1275x over previous
import functools

import jax
import jax.numpy as jnp
from jax import lax
from jax.experimental import pallas as pl
from jax.experimental.pallas import tpu as pltpu
from jax.experimental.pallas import tpu_sc as plsc

V, D, O = 1_000_000, 64, 256
B, L = 4096, 200
NC, NS = 2, 16
NW = NC * NS
NB = B // NW


CW = 256
NCH = V // CW
TV0 = V - 128
CHW = 123


def _tr_body(tT_hbm, tail_hbm, lin_hbm, in0, in1, in2, out0, out1, out2,
             isem0, isem1, isem2, osem0, osem1, osem2):
    wid = lax.axis_index("s") * NC + lax.axis_index("c")

    def fire_in(j, buf, sem):
        pltpu.async_copy(tT_hbm.at[:, pl.ds(j * CW, CW)], buf, sem)

    def wait_in(buf, sem):
        pltpu.make_async_copy(tT_hbm.at[:, pl.ds(0, CW)], buf, sem).wait()

    def fire_out(j, buf, sem):
        pltpu.async_copy(buf, lin_hbm.at[pl.ds(j * CW * D, CW * D)], sem)

    def wait_out(buf, sem):
        pltpu.make_async_copy(buf, lin_hbm.at[pl.ds(0, CW * D)], sem).wait()

    iota16 = lax.iota(jnp.int32, 16)
    dvecs = [iota16 + 16 * db for db in range(4)]
    rots = [(iota16 + s) % 16 for s in range(16)]
    outbs = [rots[s] * D + iota16 for s in range(16)]

    def transpose(inb, outb, nub):
        def ub_body(ub, carry):
            u0 = ub * 16
            c0 = ub * (16 * D)
            for db in range(4):
                vals = [plsc.load_gather(inb, [dvecs[db], rots[s] + u0])
                        for s in range(16)]
                for s in range(16):
                    plsc.store_scatter(outb, [outbs[s] + (c0 + 16 * db)],
                                       vals[s])
            return carry

        lax.fori_loop(0, nub, ub_body, 0)

    def chunk(t):
        return (wid + t * NW) % NCH

    fire_in(chunk(0), in0, isem0)
    fire_in(chunk(1), in1, isem1)
    fire_in(chunk(2), in2, isem2)

    def lt(i, carry):
        def third(t, inb, outb, isem, osem):
            wait_in(inb, isem)

            @pl.when(i > 0)
            def _():
                wait_out(outb, osem)

            transpose(inb, outb, CW // 16)
            fire_out(chunk(t), outb, osem)
            fire_in(chunk(t + 3), inb, isem)

        third(3 * i, in0, out0, isem0, osem0)
        third(3 * i + 1, in1, out1, isem1, osem1)
        third(3 * i + 2, in2, out2, isem2, osem2)
        return carry

    lax.fori_loop(0, CHW // 3, lt, 0)
    wait_in(in0, isem0)
    wait_in(in1, isem1)
    wait_in(in2, isem2)
    wait_out(out0, osem0)
    wait_out(out1, osem1)
    wait_out(out2, osem2)

    @pl.when(wid == 0)
    def _():
        pltpu.sync_copy(tail_hbm, in0.at[:, pl.ds(0, 128)])
        transpose(in0, out0, 8)
        pltpu.sync_copy(out0.at[pl.ds(0, 128 * D)],
                        lin_hbm.at[pl.ds(TV0 * D, 128 * D)])


_transpose = functools.partial(
    pl.kernel,
    mesh=plsc.VectorSubcoreMesh(core_axis_name="c", subcore_axis_name="s"),
    compiler_params=pltpu.CompilerParams(needs_layout_passes=False),
    out_type=jax.ShapeDtypeStruct((V * D,), jnp.float32),
    scratch_types=[
        pltpu.VMEM((D, CW), jnp.float32),
        pltpu.VMEM((D, CW), jnp.float32),
        pltpu.VMEM((D, CW), jnp.float32),
        pltpu.VMEM((CW * D,), jnp.float32),
        pltpu.VMEM((CW * D,), jnp.float32),
        pltpu.VMEM((CW * D,), jnp.float32),
        pltpu.SemaphoreType.DMA,
        pltpu.SemaphoreType.DMA,
        pltpu.SemaphoreType.DMA,
        pltpu.SemaphoreType.DMA,
        pltpu.SemaphoreType.DMA,
        pltpu.SemaphoreType.DMA,
    ],
)(_tr_body)


def _pool_body(tokA_hbm, tokB_hbm, table_hbm, pooled_hbm, tokA_v, tokB_v,
               buf0, buf1, buf2, buf3, out_v, sem0, sem1, sem2, sem3):
    wid = lax.axis_index("s") * NC + lax.axis_index("c")
    base = wid * NB
    pltpu.sync_copy(tokA_hbm.at[pl.ds(base, NB)], tokA_v)
    pltpu.sync_copy(tokB_hbm.at[pl.ds(base, NB)], tokB_v)

    def fire(b, buf, sem):
        pltpu.async_copy(table_hbm.at[tokA_v.at[b, pl.ds(0, 128)]],
                         buf.at[pl.ds(0, 128)], sem)
        pltpu.async_copy(table_hbm.at[tokB_v.at[b, pl.ds(0, L - 128)]],
                         buf.at[pl.ds(128, L - 128)], sem)

    def wait(buf, sem):
        pltpu.make_async_copy(table_hbm.at[pl.ds(0, L)], buf, sem).wait()

    zeros = jnp.zeros((16,), jnp.float32)

    def process(b, buf):
        def acc_body(i8, accs):
            t0 = i8 * 8
            for dt in range(8):
                accs = tuple(a + buf[t0 + dt, pl.ds(k * 16, 16)]
                             for k, a in enumerate(accs))
            return accs

        accs = lax.fori_loop(0, L // 8, acc_body, (zeros, zeros, zeros, zeros))
        for k in range(4):
            out_v[pl.ds(b * D + k * 16, 16)] = accs[k]

    bufs = (buf0, buf1, buf2, buf3)
    sems = (sem0, sem1, sem2, sem3)
    for q in range(4):
        fire(q, bufs[q], sems[q])

    def loop_body(i, carry):
        b0 = 4 * i
        for q in range(4):
            wait(bufs[q], sems[q])
            process(b0 + q, bufs[q])

            @pl.when(i < NB // 4 - 1)
            def _():
                fire(b0 + q + 4, bufs[q], sems[q])

        return carry

    lax.fori_loop(0, NB // 4, loop_body, 0)
    pltpu.sync_copy(out_v, pooled_hbm.at[pl.ds(base * D, NB * D)])


_pool = functools.partial(
    pl.kernel,
    mesh=plsc.VectorSubcoreMesh(core_axis_name="c", subcore_axis_name="s"),
    compiler_params=pltpu.CompilerParams(use_tc_tiling_on_sc=False),
    out_type=jax.ShapeDtypeStruct((B * D,), jnp.float32),
    scratch_types=[
        pltpu.VMEM((NB, 128), jnp.int32),
        pltpu.VMEM((NB, L - 128), jnp.int32),
        pltpu.VMEM((L, D), jnp.float32),
        pltpu.VMEM((L, D), jnp.float32),
        pltpu.VMEM((L, D), jnp.float32),
        pltpu.VMEM((L, D), jnp.float32),
        pltpu.VMEM((NB * D,), jnp.float32),
        pltpu.SemaphoreType.DMA,
        pltpu.SemaphoreType.DMA,
        pltpu.SemaphoreType.DMA,
        pltpu.SemaphoreType.DMA,
    ],
)(_pool_body)


def _mlp_body(x_ref, tok_ref, w1_ref, b1_ref, w2_ref, b2_ref, o_ref):
    cnt = jnp.sum((tok_ref[...] != 0).astype(jnp.float32), axis=1,
                  keepdims=True)
    x = x_ref[...] / jnp.maximum(cnt, 1.0)
    h = jnp.dot(x, w1_ref[...], preferred_element_type=jnp.float32)
    h = jnp.maximum(h + b1_ref[...], 0.0)
    p = jnp.dot(h, w2_ref[...], preferred_element_type=jnp.float32)
    p = p + b2_ref[...]
    norm = jnp.sqrt(jnp.sum(p * p, axis=-1, keepdims=True))
    o_ref[...] = p / jnp.maximum(norm, 1e-8)


BLK = 512


def _mlp(summed, tokens, W1, b1, W2, b2):
    return pl.pallas_call(
        _mlp_body,
        out_shape=jax.ShapeDtypeStruct((B, O), jnp.float32),
        grid=(B // BLK,),
        in_specs=[
            pl.BlockSpec((BLK, D), lambda i: (i, 0)),
            pl.BlockSpec((BLK, L), lambda i: (i, 0)),
            pl.BlockSpec((D, O), lambda i: (0, 0)),
            pl.BlockSpec((1, O), lambda i: (0, 0)),
            pl.BlockSpec((O, O), lambda i: (0, 0)),
            pl.BlockSpec((1, O), lambda i: (0, 0)),
        ],
        out_specs=pl.BlockSpec((BLK, O), lambda i: (i, 0)),
    )(summed, tokens, W1, b1, W2, b2)


def kernel(tokens, table, W1, b1, W2, b2):
    tT = table.T
    tail = tT[:, TV0:]
    lin = _transpose(tT, tail).reshape(V, D)
    summed = _pool(tokens[:, :128], tokens[:, 128:], lin).reshape(B, D)
    return _mlp(summed, tokens, W1, b1.reshape(1, O), W2, b2.reshape(1, O))
